# baseline (speedup 1.0000x reference)
import jax
import jax.numpy as jnp
from jax import lax
from jax.experimental import pallas as pl
from jax.experimental.pallas import tpu as pltpu
from jax.experimental.pallas import tpu_sc as plsc

VOCAB = 1000
EMB = 16
PAD = 17
NC = 2
NS = 16
NW = NC * NS
BATCH = 4096
HIST = 200
N = BATCH * HIST
PER_W = N // NW
HCHUNK = 10
NCHUNK = HIST // HCHUNK
BLK = 2 * 8 * 128
BUFSZ = HCHUNK * BLK


def _emb_kernel(x_hbm, table_hbm, out_hbm, tab_s, tab_v, idx_v, buf_v0, buf_v1,
                osem0, osem1):
  wid = lax.axis_index("s") * NC + lax.axis_index("c")

  pltpu.sync_copy(table_hbm, tab_s)
  pltpu.sync_copy(x_hbm.at[pl.ds(wid * PER_W, PER_W)], idx_v)

  @plsc.parallel_loop(0, VOCAB, unroll=4)
  def _(i):
    tab_v[pl.ds(i * PAD, EMB)] = jnp.maximum(tab_s[i], 0.0)

  lane = lax.iota(jnp.int32, 16)
  biota = lane * HIST
  cols = [jnp.full((16,), c, jnp.int32) for c in range(EMB)]

  bufs = [(buf_v0, osem0), (buf_v1, osem1)]
  pend = [[] for _ in range(NCHUNK)]
  for c in range(NCHUNK):
    buf_v, osem = bufs[c % 2]
    if c >= 2:
      for hnd in pend[c - 2]:
        hnd.wait()

    @plsc.parallel_loop(0, HCHUNK * 8, unroll=4)
    def _(q, _buf=buf_v, _h0=c * HCHUNK):
      hi = q >> 3
      g = q & 7
      gidx = biota + (g * 16 * HIST + _h0 + hi)
      iv = plsc.load_gather(idx_v, [gidx]) * PAD
      base = hi * BLK + g * 16
      for col in range(EMB):
        ev = plsc.load_gather(tab_v, [iv + cols[col]])
        _buf[pl.ds(base + col * 128, 16)] = ev

    h0 = c * HCHUNK
    for hi in range(HCHUNK):
      for ct in range(2):
        pend[c].append(pltpu.async_copy(
            buf_v.at[pl.ds((hi * 2 + ct) * 1024, 1024)],
            out_hbm.at[h0 + hi, ct, wid], osem))
  for hnd in pend[NCHUNK - 2]:
    hnd.wait()
  for hnd in pend[NCHUNK - 1]:
    hnd.wait()


@jax.jit
def _run(x_flat, table):
  mesh = plsc.VectorSubcoreMesh(core_axis_name="c", subcore_axis_name="s")
  return pl.kernel(
      _emb_kernel,
      out_type=jax.ShapeDtypeStruct((HIST, 2, NW, 1024), jnp.float32),
      mesh=mesh,
      scratch_types=[
          pltpu.VMEM((VOCAB, EMB), jnp.float32),
          pltpu.VMEM((VOCAB * PAD,), jnp.float32),
          pltpu.VMEM((PER_W,), jnp.int32),
          pltpu.VMEM((BUFSZ,), jnp.float32),
          pltpu.VMEM((BUFSZ,), jnp.float32),
          pltpu.SemaphoreType.DMA,
          pltpu.SemaphoreType.DMA,
      ],
      compiler_params=pltpu.CompilerParams(
          use_tc_tiling_on_sc=False, needs_layout_passes=False,
          disable_bounds_checks=True),
  )(x_flat, table)


def kernel(x, table):
  b, h = x.shape
  x_flat = x.reshape(-1).astype(jnp.int32)
  phys = _run(x_flat, table)
  phys5 = phys.reshape(h, 2, NW, 8, 128)
  out = phys5.transpose(2, 4, 0, 1, 3).reshape(b, h, EMB)
  return out


# speedup vs baseline: 1.0941x; 1.0941x over previous
import jax
import jax.numpy as jnp
from jax import lax
from jax.experimental import pallas as pl
from jax.experimental.pallas import tpu as pltpu
from jax.experimental.pallas import tpu_sc as plsc

VOCAB = 1000
EMB = 16
PAD = 17
NC = 2
NS = 16
NW = NC * NS
BATCH = 4096
HIST = 200
N = BATCH * HIST
PER_W = N // NW
HCHUNK = 10
NCHUNK = HIST // HCHUNK
BLK = 2 * 8 * 128
BUFSZ = HCHUNK * BLK


def _emb_kernel(x_hbm, table_hbm, out_hbm, tab_s, tab_v, idx_v, buf_v0, buf_v1,
                osem0, osem1):
  wid = lax.axis_index("s") * NC + lax.axis_index("c")

  pltpu.sync_copy(table_hbm, tab_s)
  pltpu.sync_copy(x_hbm.at[pl.ds(wid * PER_W, PER_W)], idx_v)

  @plsc.parallel_loop(0, VOCAB, unroll=4)
  def _(i):
    tab_v[pl.ds(i * PAD, EMB)] = jnp.maximum(tab_s[i], 0.0)

  lane = lax.iota(jnp.int32, 16)
  biota = lane * HIST
  cols = [jnp.full((16,), c, jnp.int32) for c in range(EMB)]

  bufs = [(buf_v0, osem0), (buf_v1, osem1)]
  pend = [[] for _ in range(NCHUNK)]
  for c in range(NCHUNK):
    buf_v, osem = bufs[c % 2]
    if c >= 2:
      for hnd in pend[c - 2]:
        hnd.wait()

    @plsc.parallel_loop(0, HCHUNK * 8, unroll=1)
    def _(q, _buf=buf_v, _h0=c * HCHUNK):
      hi = q >> 3
      g = q & 7
      gidx = biota + (g * 16 * HIST + _h0 + hi)
      iv = plsc.load_gather(idx_v, [gidx]) * PAD
      base = hi * BLK + g * 16
      for col in range(EMB):
        ev = plsc.load_gather(tab_v, [iv + cols[col]])
        _buf[pl.ds(base + col * 128, 16)] = ev

    h0 = c * HCHUNK
    for hi in range(HCHUNK):
      for ct in range(2):
        pend[c].append(pltpu.async_copy(
            buf_v.at[pl.ds((hi * 2 + ct) * 1024, 1024)],
            out_hbm.at[h0 + hi, ct, wid], osem))
  for hnd in pend[NCHUNK - 2]:
    hnd.wait()
  for hnd in pend[NCHUNK - 1]:
    hnd.wait()


@jax.jit
def _run(x_flat, table):
  mesh = plsc.VectorSubcoreMesh(core_axis_name="c", subcore_axis_name="s")
  return pl.kernel(
      _emb_kernel,
      out_type=jax.ShapeDtypeStruct((HIST, 2, NW, 1024), jnp.float32),
      mesh=mesh,
      scratch_types=[
          pltpu.VMEM((VOCAB, EMB), jnp.float32),
          pltpu.VMEM((VOCAB * PAD,), jnp.float32),
          pltpu.VMEM((PER_W,), jnp.int32),
          pltpu.VMEM((BUFSZ,), jnp.float32),
          pltpu.VMEM((BUFSZ,), jnp.float32),
          pltpu.SemaphoreType.DMA,
          pltpu.SemaphoreType.DMA,
      ],
      compiler_params=pltpu.CompilerParams(
          use_tc_tiling_on_sc=False, needs_layout_passes=False,
          disable_bounds_checks=True),
  )(x_flat, table)


def kernel(x, table):
  b, h = x.shape
  x_flat = x.reshape(-1).astype(jnp.int32)
  phys = _run(x_flat, table)
  phys5 = phys.reshape(h, 2, NW, 8, 128)
  out = phys5.transpose(2, 4, 0, 1, 3).reshape(b, h, EMB)
  return out
